# baseline (speedup 1.0000x reference)
import functools

import jax
import jax.numpy as jnp
from jax import lax
from jax.experimental import pallas as pl
from jax.experimental.pallas import tpu as pltpu
from jax.experimental.pallas import tpu_sc as plsc

N = 10000
E = 320000
D = 128
HD = D // 2
NP = 10240
E2 = E + N
NW = 32
EB = 128
NB = 84
NBD = NB // 2
CW = NB * EB
E_PAD = NW * CW
RPS = NP // 16
RC = 128

_f32 = jnp.float32
_i32 = jnp.int32


def _body_a(x_ref, w_ref, att_ref, h_ref, ats_ref, m_ref):
    i = pl.program_id(0)
    h = jnp.dot(x_ref[...], w_ref[...], preferred_element_type=_f32)
    h_ref[0, :, :] = h[:, :HD]
    h_ref[1, :, :] = h[:, HD:]
    ats = lax.dot_general(att_ref[...], h, (((1,), (1,)), ((), ())),
                          preferred_element_type=_f32)
    ats_ref[...] = ats
    cur = jnp.broadcast_to(jnp.max(ats, axis=1, keepdims=True), (8, 128))

    @pl.when(i == 0)
    def _():
        m_ref[...] = cur

    @pl.when(i > 0)
    def _():
        m_ref[...] = jnp.maximum(m_ref[...], cur)


def _proj(xp, W_att, att_pad):
    return pl.pallas_call(
        _body_a,
        grid=(NP // 1024,),
        in_specs=[
            pl.BlockSpec((1024, D), lambda i: (i, 0)),
            pl.BlockSpec((D, D), lambda i: (0, 0)),
            pl.BlockSpec((8, D), lambda i: (0, 0)),
        ],
        out_specs=[
            pl.BlockSpec((2, 1024, HD), lambda i: (0, i, 0)),
            pl.BlockSpec((8, 1024), lambda i: (0, i)),
            pl.BlockSpec((8, 128), lambda i: (0, 0)),
        ],
        out_shape=[
            jax.ShapeDtypeStruct((2, NP, HD), _f32),
            jax.ShapeDtypeStruct((8, NP), _f32),
            jax.ShapeDtypeStruct((8, 128), _f32),
        ],
    )(xp, W_att, att_pad)


def _body_b(a_s_hbm, a_d_hbm, src_hbm, dst_hbm, m_hbm,
            ex_hbm, den_hbm,
            asv, adv, srcv, dstv, exv, mv, zv, den_sh):
    c = lax.axis_index("c")
    s = lax.axis_index("s")
    wid = s * 2 + c

    for k in range(RPS // 16):
        zv[pl.ds(k * 16, 16)] = jnp.zeros((16,), _f32)
    pltpu.sync_copy(zv, den_sh.at[pl.ds(s * RPS, RPS)])

    pltpu.sync_copy(a_s_hbm, asv)
    pltpu.sync_copy(a_d_hbm, adv)
    pltpu.sync_copy(src_hbm.at[wid], srcv)
    pltpu.sync_copy(dst_hbm.at[wid], dstv)
    pltpu.sync_copy(m_hbm, mv)
    plsc.subcore_barrier()

    m = mv[...]
    base = wid * CW

    def row(j, carry):
        for k in range(EB // 16):
            si = srcv[j, pl.ds(k * 16, 16)]
            di = dstv[j, pl.ds(k * 16, 16)]
            a = plsc.load_gather(asv, [si])
            b = plsc.load_gather(adv, [di])
            e = a + b
            e = jnp.maximum(e, 0.2 * e)
            ex = jnp.exp(e - m)
            eid = base + j * EB + k * 16 + lax.iota(_i32, 16)
            ex = jnp.where(eid < E2, ex, 0.0)
            exv[j, pl.ds(k * 16, 16)] = ex
        return carry

    lax.fori_loop(0, NB, row, 0)

    def srow(j, carry):
        pltpu.sync_copy(exv.at[j], den_sh.at[dstv.at[j]], add=True)
        return carry

    lax.fori_loop(0, NB, srow, 0)
    pltpu.sync_copy(exv, ex_hbm.at[wid])
    plsc.subcore_barrier()
    pltpu.sync_copy(den_sh.at[pl.ds(s * RPS, RPS)],
                    den_hbm.at[c, pl.ds(s * RPS, RPS)])


def _attention_edges(a_s, a_d, src3, dst3, m_full):
    mesh = plsc.VectorSubcoreMesh(core_axis_name="c", subcore_axis_name="s")
    f = functools.partial(
        pl.kernel,
        out_type=[
            jax.ShapeDtypeStruct((NW, NB, EB), _f32),
            jax.ShapeDtypeStruct((2, NP), _f32),
        ],
        mesh=mesh,
        scratch_types=[
            pltpu.VMEM((NP,), _f32),
            pltpu.VMEM((NP,), _f32),
            pltpu.VMEM((NB, EB), _i32),
            pltpu.VMEM((NB, EB), _i32),
            pltpu.VMEM((NB, EB), _f32),
            pltpu.VMEM((16,), _f32),
            pltpu.VMEM((RPS,), _f32),
            pltpu.VMEM_SHARED((NP,), _f32),
        ],
        compiler_params=pltpu.CompilerParams(needs_layout_passes=False,
                                             use_tc_tiling_on_sc=False),
    )(_body_b)
    return f(a_s, a_d, src3, dst3, m_full)


def _body_c(den_ref, out_ref):
    d = den_ref[0] + den_ref[1]
    out_ref[...] = 1.0 / (d + 1e-16)


def _dinv(den):
    den2 = den.reshape(2, NP // 128, 128)
    out = pl.pallas_call(
        _body_c,
        in_specs=[pl.BlockSpec((2, NP // 128, 128), lambda: (0, 0, 0))],
        out_specs=pl.BlockSpec((NP // 128, 128), lambda: (0, 0)),
        out_shape=jax.ShapeDtypeStruct((NP // 128, 128), _f32),
    )(den2)
    return out.reshape(NP)


def _body_d(h_hbm, src_hbm, dst_hbm, ex_hbm, dinv_hbm,
            part_hbm,
            srcv, dstv, wv, dv, rows, rows1, acc_sh,
            g0, g1, s0, s1):
    c = lax.axis_index("c")
    s = lax.axis_index("s")

    def zrow(i, carry):
        for k in range(HD // 16):
            rows[i, pl.ds(k * 16, 16)] = jnp.zeros((16,), _f32)
        return carry

    lax.fori_loop(0, RC, zrow, 0)
    for t in range(RPS // RC):
        pltpu.sync_copy(rows, acc_sh.at[pl.ds(s * RPS + t * RC, RC)])

    pltpu.sync_copy(dinv_hbm.at[pl.ds(s * RPS, RPS)], dv)
    plsc.subcore_barrier()

    def scale(buf, j):
        def sg(g, carry):
            w16 = wv[j, pl.ds(g * 16, 16)]
            for el in range(16):
                e = g * 16 + el
                ws = w16[el]
                for q in range(HD // 16):
                    buf[e, pl.ds(q * 16, 16)] = buf[e, pl.ds(q * 16, 16)] * ws
            return carry

        lax.fori_loop(0, EB // 16, sg, 0)

    off = c * NP
    NT = NBD // 2
    for half in range(2):
        wid = s * 2 + half
        for qb in range(2):
            pltpu.sync_copy(src_hbm.at[wid, pl.ds(qb * NBD, NBD)], srcv)
            pltpu.sync_copy(dst_hbm.at[wid, pl.ds(qb * NBD, NBD)], dstv)
            pltpu.sync_copy(ex_hbm.at[wid, pl.ds(qb * NBD, NBD)], wv)

            def orow(j, carry):
                for k in range(EB // 16):
                    srcv[j, pl.ds(k * 16, 16)] = (
                        srcv[j, pl.ds(k * 16, 16)] + off)
                return carry

            lax.fori_loop(0, NBD, orow, 0)

            pltpu.async_copy(h_hbm.at[srcv.at[0]], rows, g0)

            def pair(t, carry):
                j0 = 2 * t
                j1 = j0 + 1
                pltpu.make_async_copy(h_hbm.at[srcv.at[j0]], rows, g0).wait()

                @pl.when(t > 0)
                def _():
                    pltpu.make_async_copy(rows1, acc_sh.at[dstv.at[j0 - 1]],
                                          s1).wait()

                pltpu.async_copy(h_hbm.at[srcv.at[j1]], rows1, g1)
                scale(rows, j0)
                pltpu.async_copy(rows, acc_sh.at[dstv.at[j0]], s0, add=True)
                pltpu.make_async_copy(h_hbm.at[srcv.at[j1]], rows1, g1).wait()

                @pl.when(t + 1 < NT)
                def _():
                    pltpu.make_async_copy(rows, acc_sh.at[dstv.at[j0]],
                                          s0).wait()
                    pltpu.async_copy(h_hbm.at[srcv.at[j0 + 2]], rows, g0)

                scale(rows1, j1)
                pltpu.async_copy(rows1, acc_sh.at[dstv.at[j1]], s1, add=True)
                return carry

            lax.fori_loop(0, NT, pair, 0)
            pltpu.make_async_copy(rows, acc_sh.at[dstv.at[NBD - 2]],
                                  s0).wait()
            pltpu.make_async_copy(rows1, acc_sh.at[dstv.at[NBD - 1]],
                                  s1).wait()
    plsc.subcore_barrier()

    for t in range(RPS // RC):
        pltpu.sync_copy(acc_sh.at[pl.ds(s * RPS + t * RC, RC)], rows)

        def srow(g, carry):
            d16 = dv[pl.ds(t * RC + g * 16, 16)]
            for el in range(16):
                r = g * 16 + el
                ws = d16[el]
                for q in range(HD // 16):
                    rows[r, pl.ds(q * 16, 16)] = rows[r, pl.ds(q * 16, 16)] * ws
            return carry

        lax.fori_loop(0, RC // 16, srow, 0)
        pltpu.sync_copy(rows, part_hbm.at[c, pl.ds(s * RPS + t * RC, RC)])


def _message_round(h2, src3, dst3, ex3, dinv):
    mesh = plsc.VectorSubcoreMesh(core_axis_name="c", subcore_axis_name="s")
    f = functools.partial(
        pl.kernel,
        out_type=jax.ShapeDtypeStruct((2, NP, HD), _f32),
        mesh=mesh,
        scratch_types=[
            pltpu.VMEM((NBD, EB), _i32),
            pltpu.VMEM((NBD, EB), _i32),
            pltpu.VMEM((NBD, EB), _f32),
            pltpu.VMEM((RPS,), _f32),
            pltpu.VMEM((EB, HD), _f32),
            pltpu.VMEM((EB, HD), _f32),
            pltpu.VMEM_SHARED((NP, HD), _f32),
            pltpu.SemaphoreType.DMA,
            pltpu.SemaphoreType.DMA,
            pltpu.SemaphoreType.DMA,
            pltpu.SemaphoreType.DMA,
        ],
        compiler_params=pltpu.CompilerParams(needs_layout_passes=False,
                                             use_tc_tiling_on_sc=False),
    )(_body_d)
    return f(h2, src3, dst3, ex3, dinv)


def _body_e(part_ref, b_ref, w_ref, x_ref, h_ref):
    p = jnp.concatenate([part_ref[0], part_ref[1]], axis=1)
    x = jnp.maximum(p + b_ref[...], 0.0)
    x_ref[...] = x
    h = jnp.dot(x, w_ref[...], preferred_element_type=_f32)
    h_ref[0, :, :] = h[:, :HD]
    h_ref[1, :, :] = h[:, HD:]


def _combine_matmul(part, b, W):
    return pl.pallas_call(
        _body_e,
        grid=(NP // 1024,),
        in_specs=[
            pl.BlockSpec((2, 1024, HD), lambda i: (0, i, 0)),
            pl.BlockSpec((D,), lambda i: (0,)),
            pl.BlockSpec((D, D), lambda i: (0, 0)),
        ],
        out_specs=[
            pl.BlockSpec((1024, D), lambda i: (i, 0)),
            pl.BlockSpec((2, 1024, HD), lambda i: (0, i, 0)),
        ],
        out_shape=[
            jax.ShapeDtypeStruct((NP, D), _f32),
            jax.ShapeDtypeStruct((2, NP, HD), _f32),
        ],
    )(part, b, W)


def _body_f(part_ref, b_ref, x_ref):
    p = jnp.concatenate([part_ref[0], part_ref[1]], axis=1)
    x_ref[...] = jnp.maximum(p + b_ref[...], 0.0)


def _combine_final(part, b):
    return pl.pallas_call(
        _body_f,
        grid=(NP // 1024,),
        in_specs=[
            pl.BlockSpec((2, 1024, HD), lambda i: (0, i, 0)),
            pl.BlockSpec((D,), lambda i: (0,)),
        ],
        out_specs=pl.BlockSpec((1024, D), lambda i: (i, 0)),
        out_shape=jax.ShapeDtypeStruct((NP, D), _f32),
    )(part, b)


def kernel(obs, edge_index, W_att, att_src, att_dst, b_att,
           W_gcn0, b_gcn0, W_gcn1, b_gcn1, W_gcn2, b_gcn2):
    x = obs.reshape(-1, D)
    xp = jnp.zeros((NP, D), _f32).at[:N].set(x)

    loop = jnp.arange(N, dtype=_i32)
    padz = jnp.zeros((E_PAD - E2,), _i32)
    src = jnp.concatenate([edge_index[0], loop, padz]).reshape(NW, NB, EB)
    dst = jnp.concatenate([edge_index[1], loop, padz]).reshape(NW, NB, EB)

    att_pad = jnp.zeros((8, D), _f32).at[0].set(att_src).at[1].set(att_dst)

    h2, ats, mx = _proj(xp, W_att, att_pad)
    h2 = h2.reshape(2 * NP, HD)
    a_s = ats[0]
    a_d = ats[1]
    m_c = jnp.maximum(mx[0, 0] + mx[1, 0], 0.0)
    m_full = jnp.full((16,), m_c, _f32)

    ex3, den = _attention_edges(a_s, a_d, src, dst, m_full)
    dinv = _dinv(den)

    part = _message_round(h2, src, dst, ex3, dinv)
    x1, h2a = _combine_matmul(part, b_att, W_gcn0)

    part = _message_round(h2a.reshape(2 * NP, HD), src, dst, ex3, dinv)
    x2, h2b = _combine_matmul(part, b_gcn0, W_gcn1)

    part = _message_round(h2b.reshape(2 * NP, HD), src, dst, ex3, dinv)
    x3, h2c = _combine_matmul(part, b_gcn1, W_gcn2)

    part = _message_round(h2c.reshape(2 * NP, HD), src, dst, ex3, dinv)
    x4 = _combine_final(part, b_gcn2)

    return x4[:N].reshape(1, N, D)


# speedup vs baseline: 2.0735x; 2.0735x over previous
import functools

import jax
import jax.numpy as jnp
from jax import lax
from jax.experimental import pallas as pl
from jax.experimental.pallas import tpu as pltpu
from jax.experimental.pallas import tpu_sc as plsc

N = 10000
E = 320000
D = 128
HD = D // 2
NP = 10240
E2 = E + N
NW = 32
EB = 64
NB = 164
CW = NB * EB
E_PAD = NW * CW
RPS = NP // 16
RC = 64

_f32 = jnp.float32
_i32 = jnp.int32


def _body_a(x_ref, w_ref, att_ref, h_ref, ats_ref, m_ref):
    i = pl.program_id(0)
    h = jnp.dot(x_ref[...], w_ref[...], preferred_element_type=_f32)
    h_ref[0, :, :] = h[:, :HD]
    h_ref[1, :, :] = h[:, HD:]
    ats = lax.dot_general(att_ref[...], h, (((1,), (1,)), ((), ())),
                          preferred_element_type=_f32)
    ats_ref[...] = ats
    cur = jnp.broadcast_to(jnp.max(ats, axis=1, keepdims=True), (8, 128))

    @pl.when(i == 0)
    def _():
        m_ref[...] = cur

    @pl.when(i > 0)
    def _():
        m_ref[...] = jnp.maximum(m_ref[...], cur)


def _proj(xp, W_att, att_pad):
    return pl.pallas_call(
        _body_a,
        grid=(NP // 1024,),
        in_specs=[
            pl.BlockSpec((1024, D), lambda i: (i, 0)),
            pl.BlockSpec((D, D), lambda i: (0, 0)),
            pl.BlockSpec((8, D), lambda i: (0, 0)),
        ],
        out_specs=[
            pl.BlockSpec((2, 1024, HD), lambda i: (0, i, 0)),
            pl.BlockSpec((8, 1024), lambda i: (0, i)),
            pl.BlockSpec((8, 128), lambda i: (0, 0)),
        ],
        out_shape=[
            jax.ShapeDtypeStruct((2, NP, HD), _f32),
            jax.ShapeDtypeStruct((8, NP), _f32),
            jax.ShapeDtypeStruct((8, 128), _f32),
        ],
    )(xp, W_att, att_pad)


def _body_b(a_s_hbm, a_d_hbm, src_hbm, dst_hbm, m_hbm,
            ex_hbm, den_hbm,
            asv, adv, srcv, dstv, exv, mv, zv, den_sh):
    c = lax.axis_index("c")
    s = lax.axis_index("s")
    wid = s * 2 + c

    for k in range(RPS // 16):
        zv[pl.ds(k * 16, 16)] = jnp.zeros((16,), _f32)
    pltpu.sync_copy(zv, den_sh.at[pl.ds(s * RPS, RPS)])

    pltpu.sync_copy(a_s_hbm, asv)
    pltpu.sync_copy(a_d_hbm, adv)
    pltpu.sync_copy(src_hbm.at[wid], srcv)
    pltpu.sync_copy(dst_hbm.at[wid], dstv)
    pltpu.sync_copy(m_hbm, mv)
    plsc.subcore_barrier()

    m = mv[...]
    base = wid * CW

    def row(j, carry):
        for k in range(EB // 16):
            si = srcv[j, pl.ds(k * 16, 16)]
            di = dstv[j, pl.ds(k * 16, 16)]
            a = plsc.load_gather(asv, [si])
            b = plsc.load_gather(adv, [di])
            e = a + b
            e = jnp.maximum(e, 0.2 * e)
            ex = jnp.exp(e - m)
            eid = base + j * EB + k * 16 + lax.iota(_i32, 16)
            ex = jnp.where(eid < E2, ex, 0.0)
            exv[j, pl.ds(k * 16, 16)] = ex
        return carry

    lax.fori_loop(0, NB, row, 0)

    def srow(j, carry):
        pltpu.sync_copy(exv.at[j], den_sh.at[dstv.at[j]], add=True)
        return carry

    lax.fori_loop(0, NB, srow, 0)
    pltpu.sync_copy(exv, ex_hbm.at[wid])
    plsc.subcore_barrier()
    pltpu.sync_copy(den_sh.at[pl.ds(s * RPS, RPS)],
                    den_hbm.at[c, pl.ds(s * RPS, RPS)])


def _attention_edges(a_s, a_d, src3, dst3, m_full):
    mesh = plsc.VectorSubcoreMesh(core_axis_name="c", subcore_axis_name="s")
    f = functools.partial(
        pl.kernel,
        out_type=[
            jax.ShapeDtypeStruct((NW, NB, EB), _f32),
            jax.ShapeDtypeStruct((2, NP), _f32),
        ],
        mesh=mesh,
        scratch_types=[
            pltpu.VMEM((NP,), _f32),
            pltpu.VMEM((NP,), _f32),
            pltpu.VMEM((NB, EB), _i32),
            pltpu.VMEM((NB, EB), _i32),
            pltpu.VMEM((NB, EB), _f32),
            pltpu.VMEM((16,), _f32),
            pltpu.VMEM((RPS,), _f32),
            pltpu.VMEM_SHARED((NP,), _f32),
        ],
        compiler_params=pltpu.CompilerParams(needs_layout_passes=False,
                                             use_tc_tiling_on_sc=False),
    )(_body_b)
    return f(a_s, a_d, src3, dst3, m_full)


def _body_c(den_ref, out_ref):
    d = den_ref[0] + den_ref[1]
    out_ref[...] = 1.0 / (d + 1e-16)


def _dinv(den):
    den2 = den.reshape(2, NP // 128, 128)
    out = pl.pallas_call(
        _body_c,
        in_specs=[pl.BlockSpec((2, NP // 128, 128), lambda: (0, 0, 0))],
        out_specs=pl.BlockSpec((NP // 128, 128), lambda: (0, 0)),
        out_shape=jax.ShapeDtypeStruct((NP // 128, 128), _f32),
    )(den2)
    return out.reshape(NP)


def _body_d(h_hbm, src_hbm, dst_hbm, ex_hbm, dinv_hbm,
            part_hbm,
            srcv, dstv, wv, dv, rows, rows1, sbuf, sbuf1, acc_sh,
            g0, g1, s0, s1):
    c = lax.axis_index("c")
    s = lax.axis_index("s")

    def zrow(i, carry):
        for k in range(HD // 16):
            sbuf[i, pl.ds(k * 16, 16)] = jnp.zeros((16,), _f32)
        return carry

    lax.fori_loop(0, RC, zrow, 0)
    for t in range(RPS // RC):
        pltpu.sync_copy(sbuf, acc_sh.at[pl.ds(s * RPS + t * RC, RC)])

    pltpu.sync_copy(dinv_hbm.at[pl.ds(s * RPS, RPS)], dv)
    plsc.subcore_barrier()

    def scale(buf, out, j):
        for g in range(EB // 16):
            w16 = wv[j, pl.ds(g * 16, 16)]
            for el in range(16):
                e = g * 16 + el
                ws = w16[el]
                for q in range(HD // 32):
                    v = buf[e, pl.ds(q * 32, 32)]
                    a, b = plsc.unpack(v, format=plsc.PackFormat.INTERLEAVED)
                    out[e, pl.ds(q * 32, 16)] = a * ws
                    out[e, pl.ds(q * 32 + 16, 16)] = b * ws

    off = c * NP
    NT = NB // 2
    for half in range(2):
        wid = s * 2 + half
        pltpu.sync_copy(src_hbm.at[wid], srcv)
        pltpu.sync_copy(dst_hbm.at[wid], dstv)
        pltpu.sync_copy(ex_hbm.at[wid], wv)

        def orow(j, carry):
            for k in range(EB // 16):
                srcv[j, pl.ds(k * 16, 16)] = srcv[j, pl.ds(k * 16, 16)] + off
            return carry

        lax.fori_loop(0, NB, orow, 0)

        pltpu.async_copy(h_hbm.at[srcv.at[0]], rows, g0)
        pltpu.async_copy(h_hbm.at[srcv.at[1]], rows1, g1)

        def pair(t, carry):
            j0 = 2 * t
            j1 = j0 + 1
            pltpu.make_async_copy(h_hbm.at[srcv.at[j0]], rows, g0).wait()

            @pl.when(t > 0)
            def _():
                pltpu.make_async_copy(sbuf, acc_sh.at[dstv.at[j0 - 2]],
                                      s0).wait()

            scale(rows, sbuf, j0)

            @pl.when(t + 1 < NT)
            def _():
                pltpu.async_copy(h_hbm.at[srcv.at[j0 + 2]], rows, g0)

            pltpu.async_copy(sbuf, acc_sh.at[dstv.at[j0]], s0, add=True)

            pltpu.make_async_copy(h_hbm.at[srcv.at[j1]], rows1, g1).wait()

            @pl.when(t > 0)
            def _():
                pltpu.make_async_copy(sbuf1, acc_sh.at[dstv.at[j1 - 2]],
                                      s1).wait()

            scale(rows1, sbuf1, j1)

            @pl.when(t + 1 < NT)
            def _():
                pltpu.async_copy(h_hbm.at[srcv.at[j1 + 2]], rows1, g1)

            pltpu.async_copy(sbuf1, acc_sh.at[dstv.at[j1]], s1, add=True)
            return carry

        lax.fori_loop(0, NT, pair, 0)
        pltpu.make_async_copy(sbuf, acc_sh.at[dstv.at[NB - 2]], s0).wait()
        pltpu.make_async_copy(sbuf1, acc_sh.at[dstv.at[NB - 1]], s1).wait()
    plsc.subcore_barrier()

    for t in range(RPS // RC):
        pltpu.sync_copy(acc_sh.at[pl.ds(s * RPS + t * RC, RC)], sbuf)

        def srow(g, carry):
            d16 = dv[pl.ds(t * RC + g * 16, 16)]
            for el in range(16):
                r = g * 16 + el
                ws = d16[el]
                for q in range(HD // 16):
                    sbuf[r, pl.ds(q * 16, 16)] = sbuf[r, pl.ds(q * 16, 16)] * ws
            return carry

        lax.fori_loop(0, RC // 16, srow, 0)
        pltpu.sync_copy(sbuf, part_hbm.at[c, pl.ds(s * RPS + t * RC, RC)])


def _message_round(h2, src3, dst3, ex3, dinv):
    mesh = plsc.VectorSubcoreMesh(core_axis_name="c", subcore_axis_name="s")
    f = functools.partial(
        pl.kernel,
        out_type=jax.ShapeDtypeStruct((2, NP, HD), _f32),
        mesh=mesh,
        scratch_types=[
            pltpu.VMEM((NB, EB), _i32),
            pltpu.VMEM((NB, EB), _i32),
            pltpu.VMEM((NB, EB), _f32),
            pltpu.VMEM((RPS,), _f32),
            pltpu.VMEM((EB, HD), jnp.bfloat16),
            pltpu.VMEM((EB, HD), jnp.bfloat16),
            pltpu.VMEM((EB, HD), _f32),
            pltpu.VMEM((EB, HD), _f32),
            pltpu.VMEM_SHARED((NP, HD), _f32),
            pltpu.SemaphoreType.DMA,
            pltpu.SemaphoreType.DMA,
            pltpu.SemaphoreType.DMA,
            pltpu.SemaphoreType.DMA,
        ],
        compiler_params=pltpu.CompilerParams(needs_layout_passes=False,
                                             use_tc_tiling_on_sc=False),
    )(_body_d)
    return f(h2, src3, dst3, ex3, dinv)


def _body_e(part_ref, b_ref, w_ref, x_ref, h_ref):
    p = jnp.concatenate([part_ref[0], part_ref[1]], axis=1)
    x = jnp.maximum(p + b_ref[...], 0.0)
    x_ref[...] = x
    h = jnp.dot(x, w_ref[...], preferred_element_type=_f32)
    h_ref[0, :, :] = h[:, :HD]
    h_ref[1, :, :] = h[:, HD:]


def _combine_matmul(part, b, W):
    return pl.pallas_call(
        _body_e,
        grid=(NP // 1024,),
        in_specs=[
            pl.BlockSpec((2, 1024, HD), lambda i: (0, i, 0)),
            pl.BlockSpec((D,), lambda i: (0,)),
            pl.BlockSpec((D, D), lambda i: (0, 0)),
        ],
        out_specs=[
            pl.BlockSpec((1024, D), lambda i: (i, 0)),
            pl.BlockSpec((2, 1024, HD), lambda i: (0, i, 0)),
        ],
        out_shape=[
            jax.ShapeDtypeStruct((NP, D), _f32),
            jax.ShapeDtypeStruct((2, NP, HD), _f32),
        ],
    )(part, b, W)


def _body_f(part_ref, b_ref, x_ref):
    p = jnp.concatenate([part_ref[0], part_ref[1]], axis=1)
    x_ref[...] = jnp.maximum(p + b_ref[...], 0.0)


def _combine_final(part, b):
    return pl.pallas_call(
        _body_f,
        grid=(NP // 1024,),
        in_specs=[
            pl.BlockSpec((2, 1024, HD), lambda i: (0, i, 0)),
            pl.BlockSpec((D,), lambda i: (0,)),
        ],
        out_specs=pl.BlockSpec((1024, D), lambda i: (i, 0)),
        out_shape=jax.ShapeDtypeStruct((NP, D), _f32),
    )(part, b)


def _h_bf(h2):
    r = h2.reshape(2, NP, HD // 32, 2, 16)
    r = r.transpose(0, 1, 2, 4, 3)
    return r.reshape(2 * NP, HD).astype(jnp.bfloat16)


def kernel(obs, edge_index, W_att, att_src, att_dst, b_att,
           W_gcn0, b_gcn0, W_gcn1, b_gcn1, W_gcn2, b_gcn2):
    x = obs.reshape(-1, D)
    xp = jnp.zeros((NP, D), _f32).at[:N].set(x)

    loop = jnp.arange(N, dtype=_i32)
    padz = jnp.zeros((E_PAD - E2,), _i32)
    src = jnp.concatenate([edge_index[0], loop, padz]).reshape(NW, NB, EB)
    dst = jnp.concatenate([edge_index[1], loop, padz]).reshape(NW, NB, EB)

    att_pad = jnp.zeros((8, D), _f32).at[0].set(att_src).at[1].set(att_dst)

    h2, ats, mx = _proj(xp, W_att, att_pad)
    h2 = _h_bf(h2)
    a_s = ats[0]
    a_d = ats[1]
    m_c = jnp.maximum(mx[0, 0] + mx[1, 0], 0.0)
    m_full = jnp.full((16,), m_c, _f32)

    ex3, den = _attention_edges(a_s, a_d, src, dst, m_full)
    dinv = _dinv(den)

    part = _message_round(h2, src, dst, ex3, dinv)
    x1, h2a = _combine_matmul(part, b_att, W_gcn0)

    part = _message_round(_h_bf(h2a), src, dst, ex3, dinv)
    x2, h2b = _combine_matmul(part, b_gcn0, W_gcn1)

    part = _message_round(_h_bf(h2b), src, dst, ex3, dinv)
    x3, h2c = _combine_matmul(part, b_gcn1, W_gcn2)

    part = _message_round(_h_bf(h2c), src, dst, ex3, dinv)
    x4 = _combine_final(part, b_gcn2)

    return x4[:N].reshape(1, N, D)
